# baseline (speedup 1.0000x reference)
import jax
from jax.experimental import pallas as pl

_BLOCK_ROWS = 4096


def _copy_body(x_ref, o_ref):
    o_ref[...] = x_ref[...]


def kernel(x):
    rows, cols = x.shape
    return pl.pallas_call(
        _copy_body,
        grid=(rows // _BLOCK_ROWS,),
        in_specs=[pl.BlockSpec((_BLOCK_ROWS, cols), lambda i: (i, 0))],
        out_specs=pl.BlockSpec((_BLOCK_ROWS, cols), lambda i: (i, 0)),
        out_shape=jax.ShapeDtypeStruct(x.shape, x.dtype),
    )(x)


# speedup vs baseline: 1.0588x; 1.0588x over previous
import jax
from jax.experimental import pallas as pl

_BLOCK_ROWS = 8192


def _copy_body(x_ref, o_ref):
    o_ref[...] = x_ref[...]


def kernel(x):
    rows, cols = x.shape
    return pl.pallas_call(
        _copy_body,
        grid=(rows // _BLOCK_ROWS,),
        in_specs=[pl.BlockSpec((_BLOCK_ROWS, cols), lambda i: (i, 0))],
        out_specs=pl.BlockSpec((_BLOCK_ROWS, cols), lambda i: (i, 0)),
        out_shape=jax.ShapeDtypeStruct(x.shape, x.dtype),
    )(x)


# speedup vs baseline: 5.5080x; 5.2019x over previous
import jax
from jax.experimental import pallas as pl

_BLOCK_COLS = 8192


def _copy_body(x_ref, o_ref):
    o_ref[...] = x_ref[...]


def kernel(x):
    rows, cols = x.shape
    xt = x.T
    out = pl.pallas_call(
        _copy_body,
        grid=(rows // _BLOCK_COLS,),
        in_specs=[pl.BlockSpec((cols, _BLOCK_COLS), lambda i: (0, i))],
        out_specs=pl.BlockSpec((cols, _BLOCK_COLS), lambda i: (0, i)),
        out_shape=jax.ShapeDtypeStruct((cols, rows), x.dtype),
    )(xt)
    return out.T
